# baseline (speedup 1.0000x reference)
import functools

import jax
import jax.numpy as jnp
from jax import lax
from jax.experimental import pallas as pl
from jax.experimental.pallas import tpu as pltpu
from jax.experimental.pallas import tpu_sc as plsc

VOCAB = 100000
EMBED = 128
BATCH = 16384

NC = 2
NS = 16
L = 16
NW = NC * NS
BPW = BATCH // NW
CHUNK = 128
NCHUNK = BPW // CHUNK


def _sc_body(center_hbm, target_hbm, co_hbm, w_hbm, ev_hbm, eu_hbm,
             vb_hbm, ub_hbm, out_hbm,
             idx_c, idx_t, co_v, w_v,
             rv0, rv1, ru0, ru1, bc_all, bt_all,
             acc_v, sem0, sem1, bsem):
    wid = lax.axis_index("s") * NC + lax.axis_index("c")
    base = wid * BPW

    pltpu.sync_copy(center_hbm.at[pl.ds(base, BPW)], idx_c)
    pltpu.sync_copy(target_hbm.at[pl.ds(base, BPW)], idx_t)
    pltpu.sync_copy(co_hbm.at[pl.ds(base, BPW)], co_v)
    pltpu.sync_copy(w_hbm.at[pl.ds(base, BPW)], w_v)

    bias_cps = (
        pltpu.async_copy(vb_hbm.at[idx_c], bc_all, bsem),
        pltpu.async_copy(ub_hbm.at[idx_t], bt_all, bsem),
    )

    rv = (rv0, rv1)
    ru = (ru0, ru1)
    sems = (sem0, sem1)

    def start_chunk(c):
        b = c % 2
        ic = idx_c.at[pl.ds(c * CHUNK, CHUNK)]
        it = idx_t.at[pl.ds(c * CHUNK, CHUNK)]
        return (
            pltpu.async_copy(ev_hbm.at[ic], rv[b], sems[b]),
            pltpu.async_copy(eu_hbm.at[it], ru[b], sems[b]),
        )

    iota = lax.iota(jnp.int32, L)
    zcol = jnp.zeros((L,), jnp.int32)

    def compute_chunk(c, loss_acc):
        b = c % 2

        def group_body(g, acc_in):
            rows = g * L + iota
            z = jnp.zeros((L,), jnp.float32)

            def k_body(k, accs):
                a0, a1 = accs
                rot = jnp.bitwise_and(iota + k, L - 1)
                for blk in range(EMBED // L):
                    col = rot + (blk * L)
                    x = plsc.load_gather(rv[b], [rows, col])
                    y = plsc.load_gather(ru[b], [rows, col])
                    if blk % 2 == 0:
                        a0 = a0 + x * y
                    else:
                        a1 = a1 + x * y
                return (a0, a1)

            a0, a1 = lax.fori_loop(0, L, k_body, (z, z))
            ip = a0 + a1
            gl = g * L
            off = c * CHUNK + gl
            boff = off + iota
            cb = plsc.load_gather(bc_all, [boff, zcol])
            tb = plsc.load_gather(bt_all, [boff, zcol])
            cg = co_v[pl.ds(off, L)]
            wg = w_v[pl.ds(off, L)]
            err = ip + cb + tb - cg
            return acc_in + wg * err * err

        return lax.fori_loop(0, CHUNK // L, group_body, loss_acc)

    loss = jnp.zeros((L,), jnp.float32)
    pending = start_chunk(0)
    for cp in bias_cps:
        cp.wait()
    for c in range(NCHUNK):
        nxt = start_chunk(c + 1) if c + 1 < NCHUNK else None
        for cp in pending:
            cp.wait()
        loss = compute_chunk(c, loss)
        pending = nxt

    acc_v[...] = loss
    pltpu.sync_copy(acc_v, out_hbm.at[wid])


def _reduce_body(x_ref, o_ref):
    o_ref[...] = jnp.sum(x_ref[...], keepdims=True).reshape(1, 1) * (1.0 / BATCH)


@jax.jit
def kernel(center_words, target_words, co_occurrences, weightings,
           embedding_v, embedding_u, v_bias, u_bias):
    center = center_words.astype(jnp.int32)
    target = target_words.astype(jnp.int32)

    mesh = plsc.VectorSubcoreMesh(core_axis_name="c", subcore_axis_name="s")
    sc = pl.kernel(
        _sc_body,
        mesh=mesh,
        out_type=jax.ShapeDtypeStruct((NW, L), jnp.float32),
        compiler_params=pltpu.CompilerParams(
            needs_layout_passes=False, use_tc_tiling_on_sc=False),
        scratch_types=[
            pltpu.VMEM((BPW,), jnp.int32),
            pltpu.VMEM((BPW,), jnp.int32),
            pltpu.VMEM((BPW,), jnp.float32),
            pltpu.VMEM((BPW,), jnp.float32),
            pltpu.VMEM((CHUNK, EMBED), jnp.float32),
            pltpu.VMEM((CHUNK, EMBED), jnp.float32),
            pltpu.VMEM((CHUNK, EMBED), jnp.float32),
            pltpu.VMEM((CHUNK, EMBED), jnp.float32),
            pltpu.VMEM((BPW, 1), jnp.float32),
            pltpu.VMEM((BPW, 1), jnp.float32),
            pltpu.VMEM((L,), jnp.float32),
            pltpu.SemaphoreType.DMA,
            pltpu.SemaphoreType.DMA,
            pltpu.SemaphoreType.DMA,
        ],
    )
    partials = sc(center, target, co_occurrences, weightings,
                  embedding_v, embedding_u, v_bias, u_bias)

    total = pl.pallas_call(
        _reduce_body,
        out_shape=jax.ShapeDtypeStruct((1, 1), jnp.float32),
    )(partials)
    return total[0, 0]


# speedup vs baseline: 5.6181x; 5.6181x over previous
import functools

import jax
import jax.numpy as jnp
from jax import lax
from jax.experimental import pallas as pl
from jax.experimental.pallas import tpu as pltpu
from jax.experimental.pallas import tpu_sc as plsc

VOCAB = 100000
EMBED = 128
BATCH = 16384

NC = 2
NS = 16
L = 16
NW = NC * NS
BPW = BATCH // NW
CHUNK = 128
NCHUNK = BPW // CHUNK


def _sc_body(center_hbm, target_hbm, co_hbm, w_hbm, ev_hbm, eu_hbm,
             bias_hbm, tgt_shift_hbm, out_hbm,
             idx_c, idx_t, co_v, w_v,
             rv0, rv1, ru0, ru1, idx_tb, bc_all, bt_all,
             acc_v, sem0, sem1, bsem):
    wid = lax.axis_index("s") * NC + lax.axis_index("c")
    base = wid * BPW

    pltpu.sync_copy(center_hbm.at[pl.ds(base, BPW)], idx_c)
    pltpu.sync_copy(target_hbm.at[pl.ds(base, BPW)], idx_t)
    pltpu.sync_copy(tgt_shift_hbm.at[pl.ds(base, BPW)], idx_tb)
    pltpu.sync_copy(co_hbm.at[pl.ds(base, BPW)], co_v)
    pltpu.sync_copy(w_hbm.at[pl.ds(base, BPW)], w_v)

    bias_cps = (
        pltpu.async_copy(bias_hbm.at[idx_c], bc_all, bsem),
        pltpu.async_copy(bias_hbm.at[idx_tb], bt_all, bsem),
    )

    rv = (rv0, rv1)
    ru = (ru0, ru1)
    sems = (sem0, sem1)

    def start_chunk(c):
        b = c % 2
        ic = idx_c.at[pl.ds(c * CHUNK, CHUNK)]
        it = idx_t.at[pl.ds(c * CHUNK, CHUNK)]
        return (
            pltpu.async_copy(ev_hbm.at[ic], rv[b], sems[b]),
            pltpu.async_copy(eu_hbm.at[it], ru[b], sems[b]),
        )

    iota = lax.iota(jnp.int32, L)
    zcol = jnp.zeros((L,), jnp.int32)

    def compute_chunk(c, loss_acc):
        b = c % 2

        def group_body(g, acc_in):
            rows = g * L + iota
            z = jnp.zeros((L,), jnp.float32)

            def k_body(k, accs):
                a0, a1 = accs
                rot = jnp.bitwise_and(iota + k, L - 1)
                for blk in range(EMBED // L):
                    col = rot + (blk * L)
                    x = plsc.load_gather(rv[b], [rows, col])
                    y = plsc.load_gather(ru[b], [rows, col])
                    if blk % 2 == 0:
                        a0 = a0 + x * y
                    else:
                        a1 = a1 + x * y
                return (a0, a1)

            a0, a1 = lax.fori_loop(0, L, k_body, (z, z))
            ip = a0 + a1
            gl = g * L
            off = c * CHUNK + gl
            cb = bc_all[pl.ds(off, L)]
            tb = bt_all[pl.ds(off, L)]
            cg = co_v[pl.ds(off, L)]
            wg = w_v[pl.ds(off, L)]
            err = ip + cb + tb - cg
            return acc_in + wg * err * err

        return lax.fori_loop(0, CHUNK // L, group_body, loss_acc)

    loss = jnp.zeros((L,), jnp.float32)
    pending = start_chunk(0)
    for cp in bias_cps:
        cp.wait()
    for c in range(NCHUNK):
        nxt = start_chunk(c + 1) if c + 1 < NCHUNK else None
        for cp in pending:
            cp.wait()
        loss = compute_chunk(c, loss)
        pending = nxt

    acc_v[...] = loss
    pltpu.sync_copy(acc_v, out_hbm.at[wid])


def _reduce_body(x_ref, o_ref):
    o_ref[...] = jnp.sum(x_ref[...], keepdims=True).reshape(1, 1) * (1.0 / BATCH)


@jax.jit
def kernel(center_words, target_words, co_occurrences, weightings,
           embedding_v, embedding_u, v_bias, u_bias):
    center = center_words.astype(jnp.int32)
    target = target_words.astype(jnp.int32)
    bias_cat = jnp.concatenate([v_bias[:, 0], u_bias[:, 0]])
    target_shift = target + VOCAB

    mesh = plsc.VectorSubcoreMesh(core_axis_name="c", subcore_axis_name="s")
    sc = pl.kernel(
        _sc_body,
        mesh=mesh,
        out_type=jax.ShapeDtypeStruct((NW, L), jnp.float32),
        compiler_params=pltpu.CompilerParams(needs_layout_passes=False),
        scratch_types=[
            pltpu.VMEM((BPW,), jnp.int32),
            pltpu.VMEM((BPW,), jnp.int32),
            pltpu.VMEM((BPW,), jnp.float32),
            pltpu.VMEM((BPW,), jnp.float32),
            pltpu.VMEM((CHUNK, EMBED), jnp.float32),
            pltpu.VMEM((CHUNK, EMBED), jnp.float32),
            pltpu.VMEM((CHUNK, EMBED), jnp.float32),
            pltpu.VMEM((CHUNK, EMBED), jnp.float32),
            pltpu.VMEM((BPW,), jnp.int32),
            pltpu.VMEM((BPW,), jnp.float32),
            pltpu.VMEM((BPW,), jnp.float32),
            pltpu.VMEM((L,), jnp.float32),
            pltpu.SemaphoreType.DMA,
            pltpu.SemaphoreType.DMA,
            pltpu.SemaphoreType.DMA,
        ],
    )
    partials = sc(center, target, co_occurrences, weightings,
                  embedding_v, embedding_u, bias_cat, target_shift)

    total = pl.pallas_call(
        _reduce_body,
        out_shape=jax.ShapeDtypeStruct((1, 1), jnp.float32),
    )(partials)
    return total[0, 0]


# speedup vs baseline: 6.0463x; 1.0762x over previous
import jax
import jax.numpy as jnp
from jax import lax
from jax.experimental import pallas as pl
from jax.experimental.pallas import tpu as pltpu
from jax.experimental.pallas import tpu_sc as plsc

VOCAB = 100000
EMBED = 128
BATCH = 16384

NC = 2
NS = 16
L = 16
NW = NC * NS
BPW = BATCH // NW
CHUNK = 64
NCHUNK = BPW // CHUNK


NBUF = 4


def _sc_body(center_hbm, target_hbm, co_hbm, w_hbm, ev_hbm, eu_hbm,
             vb_hbm, ub_hbm, out_hbm,
             idx_c, idx_t, co_v, w_v,
             rv0, rv1, rv2, rv3, ru0, ru1, ru2, ru3, bc_all, bt_all,
             acc_v, sem0, sem1, sem2, sem3, bsem, ssem):
    wid = lax.axis_index("s") * NC + lax.axis_index("c")
    base = wid * BPW

    cpi = pltpu.async_copy(center_hbm.at[pl.ds(base, BPW)], idx_c, ssem)
    cpt = pltpu.async_copy(target_hbm.at[pl.ds(base, BPW)], idx_t, ssem)
    cpi.wait()
    cpt.wait()

    rv = (rv0, rv1, rv2, rv3)
    ru = (ru0, ru1, ru2, ru3)
    sems = (sem0, sem1, sem2, sem3)

    def start_chunk(c):
        b = c % NBUF
        ic = idx_c.at[pl.ds(c * CHUNK, CHUNK)]
        it = idx_t.at[pl.ds(c * CHUNK, CHUNK)]
        return (
            pltpu.async_copy(ev_hbm.at[ic], rv[b], sems[b]),
            pltpu.async_copy(eu_hbm.at[it], ru[b], sems[b]),
        )

    inflight = [start_chunk(c) for c in range(NBUF)]

    bias_cps = (
        pltpu.async_copy(vb_hbm.at[idx_c], bc_all, bsem),
        pltpu.async_copy(ub_hbm.at[idx_t], bt_all, bsem),
    )
    scalar_cps = (
        pltpu.async_copy(co_hbm.at[pl.ds(base, BPW)], co_v, ssem),
        pltpu.async_copy(w_hbm.at[pl.ds(base, BPW)], w_v, ssem),
    )

    iota = lax.iota(jnp.int32, L)

    def compute_chunk(c, loss_acc):
        b = c % NBUF

        def group_body(g, acc_in):
            rows = g * L + iota
            z = jnp.zeros((L,), jnp.float32)

            def k_body(k, accs):
                a0, a1 = accs
                rot = jnp.bitwise_and(iota + k, L - 1)
                for blk in range(EMBED // L):
                    col = rot + (blk * L)
                    x = plsc.load_gather(rv[b], [rows, col])
                    y = plsc.load_gather(ru[b], [rows, col])
                    if blk % 2 == 0:
                        a0 = a0 + x * y
                    else:
                        a1 = a1 + x * y
                return (a0, a1)

            a0, a1 = lax.fori_loop(0, L, k_body, (z, z))
            ip = a0 + a1
            gl = g * L
            off = c * CHUNK + gl
            cb = bc_all[pl.ds(off, L)]
            tb = bt_all[pl.ds(off, L)]
            cg = co_v[pl.ds(off, L)]
            wg = w_v[pl.ds(off, L)]
            err = ip + cb + tb - cg
            return acc_in + wg * err * err

        return lax.fori_loop(0, CHUNK // L, group_body, loss_acc)

    loss = jnp.zeros((L,), jnp.float32)
    for cp in bias_cps + scalar_cps:
        cp.wait()
    for c in range(NCHUNK):
        for cp in inflight[c % NBUF]:
            cp.wait()
        loss = compute_chunk(c, loss)
        if c + NBUF < NCHUNK:
            inflight[(c + NBUF) % NBUF] = start_chunk(c + NBUF)

    acc_v[...] = loss
    pltpu.sync_copy(acc_v, out_hbm.at[wid])


def _reduce_body(x_ref, o_ref):
    o_ref[...] = jnp.sum(x_ref[...], keepdims=True).reshape(1, 1) * (1.0 / BATCH)


@jax.jit
def kernel(center_words, target_words, co_occurrences, weightings,
           embedding_v, embedding_u, v_bias, u_bias):
    center = center_words.astype(jnp.int32)
    target = target_words.astype(jnp.int32)
    vb = v_bias.reshape(VOCAB)
    ub = u_bias.reshape(VOCAB)

    mesh = plsc.VectorSubcoreMesh(core_axis_name="c", subcore_axis_name="s")
    sc = pl.kernel(
        _sc_body,
        mesh=mesh,
        out_type=jax.ShapeDtypeStruct((NW, L), jnp.float32),
        compiler_params=pltpu.CompilerParams(needs_layout_passes=False),
        scratch_types=[
            pltpu.VMEM((BPW,), jnp.int32),
            pltpu.VMEM((BPW,), jnp.int32),
            pltpu.VMEM((BPW,), jnp.float32),
            pltpu.VMEM((BPW,), jnp.float32),
            pltpu.VMEM((CHUNK, EMBED), jnp.float32),
            pltpu.VMEM((CHUNK, EMBED), jnp.float32),
            pltpu.VMEM((CHUNK, EMBED), jnp.float32),
            pltpu.VMEM((CHUNK, EMBED), jnp.float32),
            pltpu.VMEM((CHUNK, EMBED), jnp.float32),
            pltpu.VMEM((CHUNK, EMBED), jnp.float32),
            pltpu.VMEM((CHUNK, EMBED), jnp.float32),
            pltpu.VMEM((CHUNK, EMBED), jnp.float32),
            pltpu.VMEM((BPW,), jnp.float32),
            pltpu.VMEM((BPW,), jnp.float32),
            pltpu.VMEM((L,), jnp.float32),
            pltpu.SemaphoreType.DMA,
            pltpu.SemaphoreType.DMA,
            pltpu.SemaphoreType.DMA,
            pltpu.SemaphoreType.DMA,
            pltpu.SemaphoreType.DMA,
            pltpu.SemaphoreType.DMA,
        ],
    )
    partials = sc(center, target, co_occurrences, weightings,
                  embedding_v, embedding_u, vb, ub)

    total = pl.pallas_call(
        _reduce_body,
        out_shape=jax.ShapeDtypeStruct((1, 1), jnp.float32),
    )(partials)
    return total[0, 0]
